# baseline (speedup 1.0000x reference)
import functools

import jax
import jax.numpy as jnp
from jax import lax
from jax.experimental import pallas as pl
from jax.experimental.pallas import tpu as pltpu
from jax.experimental.pallas import tpu_sc as plsc

N = 10000
NPAD = 10240
E = 320000
DF = 128
H = 32
DENSE = 512
NCLS = 10
NG = 64
NSC = 2
NTILE = 16
NW = NSC * NTILE
CH = 128
NCH = 80
EW = CH * NCH
EPAD = EW * NW
NSL = NPAD // NTILE

_mesh = plsc.VectorSubcoreMesh(core_axis_name="c", subcore_axis_name="s")
_sc_params = pltpu.CompilerParams(use_tc_tiling_on_sc=False)

_Z16 = functools.partial(jnp.zeros, (16,), jnp.float32)


@functools.partial(
    pl.kernel,
    mesh=_mesh,
    out_type=jax.ShapeDtypeStruct((NSC, NPAD), jnp.float32),
    compiler_params=_sc_params,
    scratch_types=[
        pltpu.VMEM((NCH, CH), jnp.int32),
        pltpu.VMEM((CH,), jnp.float32),
        pltpu.VMEM((NSL,), jnp.float32),
        pltpu.VMEM_SHARED((NPAD,), jnp.float32),
        pltpu.SemaphoreType.DMA,
    ],
)
def _deg_kernel(edge_hbm, out_hbm, dst_v, ones_v, z_v, deg_sh, dsem):
    c = lax.axis_index("c")
    s = lax.axis_index("s")
    w = c * NTILE + s
    pltpu.sync_copy(edge_hbm.at[2, w], dst_v)

    def fill_ones(i, _):
        ones_v[pl.ds(i * 16, 16)] = jnp.ones((16,), jnp.float32)
        return 0

    lax.fori_loop(0, CH // 16, fill_ones, 0)

    def fill_zero(i, _):
        z_v[pl.ds(i * 16, 16)] = _Z16()
        return 0

    lax.fori_loop(0, NSL // 16, fill_zero, 0)
    pltpu.sync_copy(z_v, deg_sh.at[pl.ds(s * NSL, NSL)])
    plsc.subcore_barrier()

    def group(g, _):
        j0 = g * 8
        for q in range(8):
            pltpu.async_copy(ones_v, deg_sh.at[dst_v.at[j0 + q]], dsem,
                             add=True)
        for q in range(8):
            pltpu.make_async_copy(ones_v, deg_sh.at[dst_v.at[j0 + q]],
                                  dsem).wait()
        return 0

    lax.fori_loop(0, NCH // 8, group, 0)
    plsc.subcore_barrier()
    pltpu.sync_copy(deg_sh.at[pl.ds(s * NSL, NSL)],
                    out_hbm.at[c, pl.ds(s * NSL, NSL)])


@functools.partial(
    pl.kernel,
    mesh=_mesh,
    out_type=jax.ShapeDtypeStruct((NSC, NPAD, H), jnp.float32),
    compiler_params=_sc_params,
    scratch_types=[
        pltpu.VMEM((NCH, CH), jnp.int32),
        pltpu.VMEM((NCH, CH), jnp.int32),
        pltpu.VMEM((CH, H), jnp.float32),
        pltpu.VMEM((CH, H), jnp.float32),
        pltpu.VMEM((CH, H), jnp.float32),
        pltpu.VMEM((CH, H), jnp.float32),
        pltpu.VMEM((NSL, H), jnp.float32),
        pltpu.VMEM_SHARED((NPAD, H), jnp.float32),
        pltpu.SemaphoreType.DMA,
        pltpu.SemaphoreType.DMA,
        pltpu.SemaphoreType.DMA,
        pltpu.SemaphoreType.DMA,
        pltpu.SemaphoreType.DMA,
        pltpu.SemaphoreType.DMA,
        pltpu.SemaphoreType.DMA,
        pltpu.SemaphoreType.DMA,
    ],
)
def _agg_kernel(edge_hbm, hp_hbm, out_hbm,
                src_v, dst_v, rows0_v, rows1_v, rows2_v, rows3_v,
                zb_v, agg_sh,
                gs0, gs1, gs2, gs3, ss0, ss1, ss2, ss3):
    c = lax.axis_index("c")
    s = lax.axis_index("s")
    w = c * NTILE + s
    pltpu.sync_copy(edge_hbm.at[0, w], src_v)
    pltpu.sync_copy(edge_hbm.at[1, w], dst_v)

    def fill_zero(i, _):
        zb_v[i, pl.ds(0, 16)] = _Z16()
        zb_v[i, pl.ds(16, 16)] = _Z16()
        return 0

    lax.fori_loop(0, NSL, fill_zero, 0)
    pltpu.sync_copy(zb_v, agg_sh.at[pl.ds(s * NSL, NSL)])
    plsc.subcore_barrier()

    rows = (rows0_v, rows1_v, rows2_v, rows3_v)
    gsem = (gs0, gs1, gs2, gs3)
    ssem = (ss0, ss1, ss2, ss3)
    gsrc = (hp_hbm, hp_hbm, hp_hbm, hp_hbm)
    for p in range(3):
        pltpu.async_copy(gsrc[p].at[src_v.at[p]], rows[p], gsem[p])

    def quad(i, _):
        j0 = i * 4
        for b in range(4):
            j = j0 + b
            nxt = j + 3
            nb = (b + 3) % 4

            @pl.when(nxt < NCH)
            def _():
                @pl.when(j >= 1)
                def _():
                    pltpu.make_async_copy(rows[nb], agg_sh.at[dst_v.at[j - 1]],
                                          ssem[nb]).wait()
                pltpu.async_copy(gsrc[nb].at[src_v.at[nxt]], rows[nb], gsem[nb])

            pltpu.make_async_copy(gsrc[b].at[src_v.at[j]],
                                  rows[b], gsem[b]).wait()
            pltpu.async_copy(rows[b], agg_sh.at[dst_v.at[j]], ssem[b],
                             add=True)
        return 0

    lax.fori_loop(0, NCH // 4, quad, 0)
    for b in range(4):
        j = NCH - 4 + b
        pltpu.make_async_copy(rows[b], agg_sh.at[dst_v.at[j]],
                              ssem[b]).wait()
    plsc.subcore_barrier()
    pltpu.sync_copy(agg_sh.at[pl.ds(s * NSL, NSL)],
                    out_hbm.at[c, pl.ds(s * NSL, NSL)])


KP = NPAD // 4


def _scale_cols(degp4):
    deg4 = degp4[0] + degp4[1]
    kpb = deg4.shape[1]
    r = lax.broadcasted_iota(jnp.int32, (4, 4), 0)
    c = lax.broadcasted_iota(jnp.int32, (4, 4), 1)
    eye4 = (r == c).astype(jnp.float32)
    deg_n = lax.dot_general(deg4, eye4, (((0,), (0,)), ((), ())),
                            preferred_element_type=jnp.float32)
    return lax.rsqrt(jnp.maximum(deg_n, 1.0))


KPB = KP // 4


def _hprime_body(x_ref, w1_ref, degp_ref, out_ref):
    scale = _scale_cols(degp_ref[...])
    for o in range(4):
        x_o = x_ref[:, o * DF:(o + 1) * DF]
        h_o = jnp.dot(x_o, w1_ref[...],
                      preferred_element_type=jnp.float32)
        out_ref[:, o * H:(o + 1) * H] = h_o * scale[:, o:o + 1]


def _hprime_call(x_r, w1, degp4):
    return pl.pallas_call(
        _hprime_body,
        grid=(4,),
        in_specs=[
            pl.BlockSpec((KPB, 4 * DF), lambda i: (i, 0)),
            pl.BlockSpec((DF, H), lambda i: (0, 0)),
            pl.BlockSpec((NSC, 4, KPB), lambda i: (0, 0, i)),
        ],
        out_specs=pl.BlockSpec((KPB, 4 * H), lambda i: (i, 0)),
        out_shape=jax.ShapeDtypeStruct((KP, 4 * H), jnp.float32),
    )(x_r, w1, degp4)


def _tail_body(aggp_ref, degp_ref, b1_ref, ind_ref, w2_ref, b2_ref,
               w3_ref, b3_ref, out_ref):
    scale = _scale_cols(degp_ref[...])
    agg = aggp_ref[0] + aggp_ref[1]
    gids = lax.broadcasted_iota(jnp.int32, (NG, KP), 0)
    sums = jnp.zeros((NG, H), jnp.float32)
    counts = jnp.zeros((NG, 1), jnp.float32)
    for o in range(4):
        h2_o = jnp.maximum(
            agg[:, o * H:(o + 1) * H] * scale[:, o:o + 1] + b1_ref[...], 0.0)
        onehot_o = (ind_ref[o:o + 1, :] == gids).astype(jnp.float32)
        sums = sums + jnp.dot(onehot_o, h2_o,
                              preferred_element_type=jnp.float32)
        counts = counts + jnp.sum(onehot_o, axis=1, keepdims=True)
    pooled = sums / jnp.maximum(counts, 1.0)
    z = jnp.maximum(
        jnp.dot(pooled, w2_ref[...], preferred_element_type=jnp.float32)
        + b2_ref[...], 0.0)
    logits = (jnp.dot(z, w3_ref[...], preferred_element_type=jnp.float32)
              + b3_ref[...])
    m = jnp.max(logits, axis=-1, keepdims=True)
    ex = jnp.exp(logits - m)
    out_ref[...] = ex / jnp.sum(ex, axis=-1, keepdims=True)


def _tail_call(aggp, degp_r, b1, ind, w2, b2, w3, b3):
    return pl.pallas_call(
        _tail_body,
        out_shape=jax.ShapeDtypeStruct((NG, NCLS), jnp.float32),
    )(aggp, degp_r, b1, ind, w2, b2, w3, b3)


def kernel(x, edge_index, node_indicator, W1, b1, W2, b2, W3, b3):
    ei = edge_index.astype(jnp.int32)
    npe = EPAD - E
    pad_src = (jnp.arange(npe, dtype=jnp.int32) * 97) % N
    pad_dst = N + (jnp.arange(npe, dtype=jnp.int32) % (NPAD - N))
    src_p = jnp.concatenate([ei[0], pad_src])
    dst_p = jnp.concatenate([ei[1], pad_dst])
    dst4_p = (dst_p % 4) * KP + dst_p // 4
    edge4 = jnp.stack([src_p, dst_p, dst4_p]).reshape(3, NW, NCH, CH)

    degp = _deg_kernel(edge4)
    degp4 = degp.reshape(NSC, 4, KP)
    x_r = jnp.pad(x, ((0, NPAD - N), (0, 0))).reshape(KP, 4 * DF)
    hp_p = _hprime_call(x_r, W1, degp4)
    hp = hp_p.reshape(NPAD, H)
    aggp = _agg_kernel(edge4, hp)
    aggp_r = aggp.reshape(NSC, KP, 4 * H)

    ind = jnp.concatenate(
        [node_indicator.astype(jnp.int32),
         jnp.full((NPAD - N,), NG, jnp.int32)])
    ind4 = ind.reshape(KP, 4).T
    return _tail_call(aggp_r, degp4, b1.reshape(1, H), ind4,
                      W2, b2.reshape(1, DENSE), W3, b3.reshape(1, NCLS))


# speedup vs baseline: 1.1581x; 1.1581x over previous
import functools

import jax
import jax.numpy as jnp
from jax import lax
from jax.experimental import pallas as pl
from jax.experimental.pallas import tpu as pltpu
from jax.experimental.pallas import tpu_sc as plsc

N = 10000
NPAD = 10240
E = 320000
DF = 128
H = 32
DENSE = 512
NCLS = 10
NG = 64
NSC = 2
NTILE = 16
NW = NSC * NTILE
CH = 128
NCH = 80
EW = CH * NCH
EPAD = EW * NW
NSL = NPAD // NTILE

_mesh = plsc.VectorSubcoreMesh(core_axis_name="c", subcore_axis_name="s")
_sc_params = pltpu.CompilerParams(use_tc_tiling_on_sc=False)

_Z16 = functools.partial(jnp.zeros, (16,), jnp.float32)


@functools.partial(
    pl.kernel,
    mesh=_mesh,
    out_type=jax.ShapeDtypeStruct((NSC, NPAD), jnp.float32),
    compiler_params=_sc_params,
    scratch_types=[
        pltpu.VMEM((NCH, CH), jnp.int32),
        pltpu.VMEM((CH,), jnp.float32),
        pltpu.VMEM((NSL,), jnp.float32),
        pltpu.VMEM_SHARED((NPAD,), jnp.float32),
        pltpu.SemaphoreType.DMA,
    ],
)
def _deg_kernel(edge_hbm, out_hbm, dst_v, ones_v, z_v, deg_sh, dsem):
    c = lax.axis_index("c")
    s = lax.axis_index("s")
    w = c * NTILE + s
    pltpu.sync_copy(edge_hbm.at[1, w], dst_v)

    def permute(i, _):
        j = i >> 3
        kk = i & 7
        v = dst_v[j, pl.ds(kk * 16, 16)]
        dst_v[j, pl.ds(kk * 16, 16)] = (v & 3) * KP + (v >> 2)
        return 0

    lax.fori_loop(0, NCH * CH // 16, permute, 0)

    def fill_ones(i, _):
        ones_v[pl.ds(i * 16, 16)] = jnp.ones((16,), jnp.float32)
        return 0

    lax.fori_loop(0, CH // 16, fill_ones, 0)

    def fill_zero(i, _):
        z_v[pl.ds(i * 16, 16)] = _Z16()
        return 0

    lax.fori_loop(0, NSL // 16, fill_zero, 0)
    pltpu.sync_copy(z_v, deg_sh.at[pl.ds(s * NSL, NSL)])
    plsc.subcore_barrier()

    def group(g, _):
        j0 = g * 8
        for q in range(8):
            pltpu.async_copy(ones_v, deg_sh.at[dst_v.at[j0 + q]], dsem,
                             add=True)
        for q in range(8):
            pltpu.make_async_copy(ones_v, deg_sh.at[dst_v.at[j0 + q]],
                                  dsem).wait()
        return 0

    lax.fori_loop(0, NCH // 8, group, 0)
    plsc.subcore_barrier()
    pltpu.sync_copy(deg_sh.at[pl.ds(s * NSL, NSL)],
                    out_hbm.at[c, pl.ds(s * NSL, NSL)])


@functools.partial(
    pl.kernel,
    mesh=_mesh,
    out_type=jax.ShapeDtypeStruct((NSC, NPAD, H), jnp.float32),
    compiler_params=_sc_params,
    scratch_types=[
        pltpu.VMEM((NCH, CH), jnp.int32),
        pltpu.VMEM((NCH, CH), jnp.int32),
        pltpu.VMEM((CH, H), jnp.float32),
        pltpu.VMEM((CH, H), jnp.float32),
        pltpu.VMEM((CH, H), jnp.float32),
        pltpu.VMEM((CH, H), jnp.float32),
        pltpu.VMEM((NSL, H), jnp.float32),
        pltpu.VMEM_SHARED((NPAD, H), jnp.float32),
        pltpu.SemaphoreType.DMA,
        pltpu.SemaphoreType.DMA,
        pltpu.SemaphoreType.DMA,
        pltpu.SemaphoreType.DMA,
        pltpu.SemaphoreType.DMA,
        pltpu.SemaphoreType.DMA,
        pltpu.SemaphoreType.DMA,
        pltpu.SemaphoreType.DMA,
    ],
)
def _agg_kernel(edge_hbm, hp_hbm, out_hbm,
                src_v, dst_v, rows0_v, rows1_v, rows2_v, rows3_v,
                zb_v, agg_sh,
                gs0, gs1, gs2, gs3, ss0, ss1, ss2, ss3):
    c = lax.axis_index("c")
    s = lax.axis_index("s")
    w = c * NTILE + s
    pltpu.sync_copy(edge_hbm.at[0, w], src_v)
    pltpu.sync_copy(edge_hbm.at[1, w], dst_v)

    def fill_zero(i, _):
        zb_v[i, pl.ds(0, 16)] = _Z16()
        zb_v[i, pl.ds(16, 16)] = _Z16()
        return 0

    lax.fori_loop(0, NSL, fill_zero, 0)
    pltpu.sync_copy(zb_v, agg_sh.at[pl.ds(s * NSL, NSL)])
    plsc.subcore_barrier()

    rows = (rows0_v, rows1_v, rows2_v, rows3_v)
    gsem = (gs0, gs1, gs2, gs3)
    ssem = (ss0, ss1, ss2, ss3)
    gsrc = (hp_hbm, hp_hbm, hp_hbm, hp_hbm)
    for p in range(3):
        pltpu.async_copy(gsrc[p].at[src_v.at[p]], rows[p], gsem[p])

    def quad(i, _):
        j0 = i * 4
        for b in range(4):
            j = j0 + b
            nxt = j + 3
            nb = (b + 3) % 4

            @pl.when(nxt < NCH)
            def _():
                @pl.when(j >= 1)
                def _():
                    pltpu.make_async_copy(rows[nb], agg_sh.at[dst_v.at[j - 1]],
                                          ssem[nb]).wait()
                pltpu.async_copy(gsrc[nb].at[src_v.at[nxt]], rows[nb], gsem[nb])

            pltpu.make_async_copy(gsrc[b].at[src_v.at[j]],
                                  rows[b], gsem[b]).wait()
            pltpu.async_copy(rows[b], agg_sh.at[dst_v.at[j]], ssem[b],
                             add=True)
        return 0

    lax.fori_loop(0, NCH // 4, quad, 0)
    for b in range(4):
        j = NCH - 4 + b
        pltpu.make_async_copy(rows[b], agg_sh.at[dst_v.at[j]],
                              ssem[b]).wait()
    plsc.subcore_barrier()
    pltpu.sync_copy(agg_sh.at[pl.ds(s * NSL, NSL)],
                    out_hbm.at[c, pl.ds(s * NSL, NSL)])


KP = NPAD // 4


def _scale_cols(degp4):
    deg4 = degp4[0] + degp4[1]
    kpb = deg4.shape[1]
    r = lax.broadcasted_iota(jnp.int32, (4, 4), 0)
    c = lax.broadcasted_iota(jnp.int32, (4, 4), 1)
    eye4 = (r == c).astype(jnp.float32)
    deg_n = lax.dot_general(deg4, eye4, (((0,), (0,)), ((), ())),
                            preferred_element_type=jnp.float32)
    return lax.rsqrt(jnp.maximum(deg_n, 1.0))


KPB = KP // 4


def _hprime_body(x_ref, w1_ref, degp_ref, out_ref):
    scale = _scale_cols(degp_ref[...])
    for o in range(4):
        x_o = x_ref[:, o * DF:(o + 1) * DF]
        h_o = jnp.dot(x_o, w1_ref[...],
                      preferred_element_type=jnp.float32)
        out_ref[:, o * H:(o + 1) * H] = h_o * scale[:, o:o + 1]


def _hprime_call(x_r, w1, degp4):
    return pl.pallas_call(
        _hprime_body,
        grid=(4,),
        in_specs=[
            pl.BlockSpec((KPB, 4 * DF), lambda i: (i, 0)),
            pl.BlockSpec((DF, H), lambda i: (0, 0)),
            pl.BlockSpec((NSC, 4, KPB), lambda i: (0, 0, i)),
        ],
        out_specs=pl.BlockSpec((KPB, 4 * H), lambda i: (i, 0)),
        out_shape=jax.ShapeDtypeStruct((KP, 4 * H), jnp.float32),
    )(x_r, w1, degp4)


def _tail_body(aggp_ref, degp_ref, b1_ref, ind_ref, w2_ref, b2_ref,
               w3_ref, b3_ref, out_ref):
    scale = _scale_cols(degp_ref[...])
    agg = aggp_ref[0] + aggp_ref[1]
    gids = lax.broadcasted_iota(jnp.int32, (NG, KP), 0)
    sums = jnp.zeros((NG, H), jnp.float32)
    counts = jnp.zeros((NG, 1), jnp.float32)
    for o in range(4):
        h2_o = jnp.maximum(
            agg[:, o * H:(o + 1) * H] * scale[:, o:o + 1] + b1_ref[...], 0.0)
        onehot_o = (ind_ref[o:o + 1, :] == gids).astype(jnp.float32)
        sums = sums + jnp.dot(onehot_o, h2_o,
                              preferred_element_type=jnp.float32)
        counts = counts + jnp.sum(onehot_o, axis=1, keepdims=True)
    pooled = sums / jnp.maximum(counts, 1.0)
    z = jnp.maximum(
        jnp.dot(pooled, w2_ref[...], preferred_element_type=jnp.float32)
        + b2_ref[...], 0.0)
    logits = (jnp.dot(z, w3_ref[...], preferred_element_type=jnp.float32)
              + b3_ref[...])
    m = jnp.max(logits, axis=-1, keepdims=True)
    ex = jnp.exp(logits - m)
    out_ref[...] = ex / jnp.sum(ex, axis=-1, keepdims=True)


def _tail_call(aggp, degp_r, b1, ind, w2, b2, w3, b3):
    return pl.pallas_call(
        _tail_body,
        out_shape=jax.ShapeDtypeStruct((NG, NCLS), jnp.float32),
    )(aggp, degp_r, b1, ind, w2, b2, w3, b3)


def kernel(x, edge_index, node_indicator, W1, b1, W2, b2, W3, b3):
    ei = edge_index.astype(jnp.int32)
    npe = EPAD - E
    pad_src = (jnp.arange(npe, dtype=jnp.int32) * 97) % N
    pad_dst = N + (jnp.arange(npe, dtype=jnp.int32) % (NPAD - N))
    edge4 = jnp.concatenate(
        [ei, jnp.stack([pad_src, pad_dst])], axis=1).reshape(2, NW, NCH, CH)

    degp = _deg_kernel(edge4)
    degp4 = degp.reshape(NSC, 4, KP)
    x_r = jnp.pad(x, ((0, NPAD - N), (0, 0))).reshape(KP, 4 * DF)
    hp_p = _hprime_call(x_r, W1, degp4)
    hp = hp_p.reshape(NPAD, H)
    aggp = _agg_kernel(edge4, hp)
    aggp_r = aggp.reshape(NSC, KP, 4 * H)

    ind = jnp.concatenate(
        [node_indicator.astype(jnp.int32),
         jnp.full((NPAD - N,), NG, jnp.int32)])
    ind4 = ind.reshape(KP, 4).T
    return _tail_call(aggp_r, degp4, b1.reshape(1, H), ind4,
                      W2, b2.reshape(1, DENSE), W3, b3.reshape(1, NCLS))


# speedup vs baseline: 1.1991x; 1.0354x over previous
import functools

import jax
import jax.numpy as jnp
from jax import lax
from jax.experimental import pallas as pl
from jax.experimental.pallas import tpu as pltpu
from jax.experimental.pallas import tpu_sc as plsc

N = 10000
NPAD = 10240
E = 320000
DF = 128
H = 32
DENSE = 512
NCLS = 10
NG = 64
NSC = 2
NTILE = 16
NW = NSC * NTILE
CH = 128
NCH = 80
EW = CH * NCH
EPAD = EW * NW
NSL = NPAD // NTILE

_mesh = plsc.VectorSubcoreMesh(core_axis_name="c", subcore_axis_name="s")
_sc_params = pltpu.CompilerParams(use_tc_tiling_on_sc=False)

_Z16 = functools.partial(jnp.zeros, (16,), jnp.float32)


@functools.partial(
    pl.kernel,
    mesh=_mesh,
    out_type=jax.ShapeDtypeStruct((NSC, NPAD), jnp.float32),
    compiler_params=_sc_params,
    scratch_types=[
        pltpu.VMEM((NCH, CH), jnp.int32),
        pltpu.VMEM((CH,), jnp.float32),
        pltpu.VMEM((NSL,), jnp.float32),
        pltpu.VMEM_SHARED((NPAD,), jnp.float32),
        pltpu.SemaphoreType.DMA,
    ],
)
def _deg_kernel(edge_hbm, out_hbm, dst_v, ones_v, z_v, deg_sh, dsem):
    c = lax.axis_index("c")
    s = lax.axis_index("s")
    w = c * NTILE + s
    pltpu.sync_copy(edge_hbm.at[1, w], dst_v)

    def permute(i, _):
        j = i >> 3
        kk = i & 7
        v = dst_v[j, pl.ds(kk * 16, 16)]
        dst_v[j, pl.ds(kk * 16, 16)] = (v & 3) * KP + (v >> 2)
        return 0

    lax.fori_loop(0, NCH * CH // 16, permute, 0)

    def fill_ones(i, _):
        ones_v[pl.ds(i * 16, 16)] = jnp.ones((16,), jnp.float32)
        return 0

    lax.fori_loop(0, CH // 16, fill_ones, 0)

    def fill_zero(i, _):
        z_v[pl.ds(i * 16, 16)] = _Z16()
        return 0

    lax.fori_loop(0, NSL // 16, fill_zero, 0)
    pltpu.sync_copy(z_v, deg_sh.at[pl.ds(s * NSL, NSL)])
    plsc.subcore_barrier()

    def group(g, _):
        j0 = g * 8
        for q in range(8):
            pltpu.async_copy(ones_v, deg_sh.at[dst_v.at[j0 + q]], dsem,
                             add=True)
        for q in range(8):
            pltpu.make_async_copy(ones_v, deg_sh.at[dst_v.at[j0 + q]],
                                  dsem).wait()
        return 0

    lax.fori_loop(0, NCH // 8, group, 0)
    plsc.subcore_barrier()
    pltpu.sync_copy(deg_sh.at[pl.ds(s * NSL, NSL)],
                    out_hbm.at[c, pl.ds(s * NSL, NSL)])


@functools.partial(
    pl.kernel,
    mesh=_mesh,
    out_type=jax.ShapeDtypeStruct((NSC, NPAD, H), jnp.float32),
    compiler_params=_sc_params,
    scratch_types=[
        pltpu.VMEM((NCH, CH), jnp.int32),
        pltpu.VMEM((NCH, CH), jnp.int32),
        pltpu.VMEM((CH, H), jnp.float32),
        pltpu.VMEM((CH, H), jnp.float32),
        pltpu.VMEM((CH, H), jnp.float32),
        pltpu.VMEM((CH, H), jnp.float32),
        pltpu.VMEM((NSL, H), jnp.float32),
        pltpu.VMEM_SHARED((NPAD, H), jnp.float32),
        pltpu.SemaphoreType.DMA,
        pltpu.SemaphoreType.DMA,
        pltpu.SemaphoreType.DMA,
        pltpu.SemaphoreType.DMA,
        pltpu.SemaphoreType.DMA,
        pltpu.SemaphoreType.DMA,
        pltpu.SemaphoreType.DMA,
        pltpu.SemaphoreType.DMA,
    ],
)
def _agg_kernel(edge_hbm, hp_hbm, out_hbm,
                src_v, dst_v, rows0_v, rows1_v, rows2_v, rows3_v,
                zb_v, agg_sh,
                gs0, gs1, gs2, gs3, ss0, ss1, ss2, ss3):
    c = lax.axis_index("c")
    s = lax.axis_index("s")
    w = c * NTILE + s
    pltpu.sync_copy(edge_hbm.at[0, w], src_v)
    pltpu.sync_copy(edge_hbm.at[1, w], dst_v)

    def fill_zero(i, _):
        zb_v[i, pl.ds(0, 16)] = _Z16()
        zb_v[i, pl.ds(16, 16)] = _Z16()
        return 0

    lax.fori_loop(0, NSL, fill_zero, 0)
    pltpu.sync_copy(zb_v, agg_sh.at[pl.ds(s * NSL, NSL)])
    plsc.subcore_barrier()

    rows = (rows0_v, rows1_v, rows2_v, rows3_v)
    gsem = (gs0, gs1, gs2, gs3)
    ssem = (ss0, ss1, ss2, ss3)
    gsrc = (hp_hbm, hp_hbm, hp_hbm, hp_hbm)
    for p in range(3):
        pltpu.async_copy(gsrc[p].at[src_v.at[p]], rows[p], gsem[p])

    def quad(i, _):
        j0 = i * 4
        for b in range(4):
            j = j0 + b
            nxt = j + 3
            nb = (b + 3) % 4

            @pl.when(nxt < NCH)
            def _():
                @pl.when(j >= 1)
                def _():
                    pltpu.make_async_copy(rows[nb], agg_sh.at[dst_v.at[j - 1]],
                                          ssem[nb]).wait()
                pltpu.async_copy(gsrc[nb].at[src_v.at[nxt]], rows[nb], gsem[nb])

            pltpu.make_async_copy(gsrc[b].at[src_v.at[j]],
                                  rows[b], gsem[b]).wait()
            pltpu.async_copy(rows[b], agg_sh.at[dst_v.at[j]], ssem[b],
                             add=True)
        return 0

    lax.fori_loop(0, NCH // 4, quad, 0)
    for b in range(4):
        j = NCH - 4 + b
        pltpu.make_async_copy(rows[b], agg_sh.at[dst_v.at[j]],
                              ssem[b]).wait()
    plsc.subcore_barrier()
    pltpu.sync_copy(agg_sh.at[pl.ds(s * NSL, NSL)],
                    out_hbm.at[c, pl.ds(s * NSL, NSL)])


KP = NPAD // 4


def _scale_cols(degp4):
    deg4 = degp4[0] + degp4[1]
    kpb = deg4.shape[1]
    r = lax.broadcasted_iota(jnp.int32, (4, 4), 0)
    c = lax.broadcasted_iota(jnp.int32, (4, 4), 1)
    eye4 = (r == c).astype(jnp.float32)
    deg_n = lax.dot_general(deg4, eye4, (((0,), (0,)), ((), ())),
                            preferred_element_type=jnp.float32)
    return lax.rsqrt(jnp.maximum(deg_n, 1.0))


KPB = KP // 4


def _hmat_body(x_ref, w1_ref, out_ref):
    for o in range(4):
        x_o = x_ref[:, o * DF:(o + 1) * DF]
        out_ref[:, o * H:(o + 1) * H] = jnp.dot(
            x_o, w1_ref[...], preferred_element_type=jnp.float32)


def _hmat_call(x_r, w1):
    return pl.pallas_call(
        _hmat_body,
        grid=(4,),
        in_specs=[
            pl.BlockSpec((KPB, 4 * DF), lambda i: (i, 0)),
            pl.BlockSpec((DF, H), lambda i: (0, 0)),
        ],
        out_specs=pl.BlockSpec((KPB, 4 * H), lambda i: (i, 0)),
        out_shape=jax.ShapeDtypeStruct((KP, 4 * H), jnp.float32),
    )(x_r, w1)


def _hscale_body(h_ref, degp_ref, out_ref):
    scale = _scale_cols(degp_ref[...])
    for o in range(4):
        out_ref[:, o * H:(o + 1) * H] = (
            h_ref[:, o * H:(o + 1) * H] * scale[:, o:o + 1])


def _hscale_call(h_p, degp4):
    return pl.pallas_call(
        _hscale_body,
        out_shape=jax.ShapeDtypeStruct((KP, 4 * H), jnp.float32),
    )(h_p, degp4)


def _tail_body(aggp_ref, degp_ref, b1_ref, ind_ref, w2_ref, b2_ref,
               w3_ref, b3_ref, out_ref):
    scale = _scale_cols(degp_ref[...])
    agg = aggp_ref[0] + aggp_ref[1]
    gids = lax.broadcasted_iota(jnp.int32, (NG, KP), 0)
    sums = jnp.zeros((NG, H), jnp.float32)
    counts = jnp.zeros((NG, 1), jnp.float32)
    for o in range(4):
        h2_o = jnp.maximum(
            agg[:, o * H:(o + 1) * H] * scale[:, o:o + 1] + b1_ref[...], 0.0)
        onehot_o = (ind_ref[o:o + 1, :] == gids).astype(jnp.float32)
        sums = sums + jnp.dot(onehot_o, h2_o,
                              preferred_element_type=jnp.float32)
        counts = counts + jnp.sum(onehot_o, axis=1, keepdims=True)
    pooled = sums / jnp.maximum(counts, 1.0)
    z = jnp.maximum(
        jnp.dot(pooled, w2_ref[...], preferred_element_type=jnp.float32)
        + b2_ref[...], 0.0)
    logits = (jnp.dot(z, w3_ref[...], preferred_element_type=jnp.float32)
              + b3_ref[...])
    m = jnp.max(logits, axis=-1, keepdims=True)
    ex = jnp.exp(logits - m)
    out_ref[...] = ex / jnp.sum(ex, axis=-1, keepdims=True)


def _tail_call(aggp, degp_r, b1, ind, w2, b2, w3, b3):
    return pl.pallas_call(
        _tail_body,
        out_shape=jax.ShapeDtypeStruct((NG, NCLS), jnp.float32),
    )(aggp, degp_r, b1, ind, w2, b2, w3, b3)


def kernel(x, edge_index, node_indicator, W1, b1, W2, b2, W3, b3):
    ei = edge_index.astype(jnp.int32)
    npe = EPAD - E
    pad_src = (jnp.arange(npe, dtype=jnp.int32) * 97) % N
    pad_dst = N + (jnp.arange(npe, dtype=jnp.int32) % (NPAD - N))
    edge4 = jnp.concatenate(
        [ei, jnp.stack([pad_src, pad_dst])], axis=1).reshape(2, NW, NCH, CH)

    degp = _deg_kernel(edge4)
    degp4 = degp.reshape(NSC, 4, KP)
    x_r = jnp.pad(x, ((0, NPAD - N), (0, 0))).reshape(KP, 4 * DF)
    h_p = _hmat_call(x_r, W1)
    hp_p = _hscale_call(h_p, degp4)
    hp = hp_p.reshape(NPAD, H)
    aggp = _agg_kernel(edge4, hp)
    aggp_r = aggp.reshape(NSC, KP, 4 * H)

    ind = jnp.concatenate(
        [node_indicator.astype(jnp.int32),
         jnp.full((NPAD - N,), NG, jnp.int32)])
    ind4 = ind.reshape(KP, 4).T
    return _tail_call(aggp_r, degp4, b1.reshape(1, H), ind4,
                      W2, b2.reshape(1, DENSE), W3, b3.reshape(1, NCLS))


# speedup vs baseline: 1.2478x; 1.0406x over previous
import functools

import jax
import jax.numpy as jnp
from jax import lax
from jax.experimental import pallas as pl
from jax.experimental.pallas import tpu as pltpu
from jax.experimental.pallas import tpu_sc as plsc

N = 10000
NPAD = 10240
E = 320000
DF = 128
H = 32
DENSE = 512
NCLS = 10
NG = 64
NSC = 2
NTILE = 16
NW = NSC * NTILE
CH = 128
NCH = 80
EW = CH * NCH
EPAD = EW * NW
NSL = NPAD // NTILE

_mesh = plsc.VectorSubcoreMesh(core_axis_name="c", subcore_axis_name="s")
_sc_params = pltpu.CompilerParams(use_tc_tiling_on_sc=False)

_Z16 = functools.partial(jnp.zeros, (16,), jnp.float32)


@functools.partial(
    pl.kernel,
    mesh=_mesh,
    out_type=jax.ShapeDtypeStruct((NSC, NPAD), jnp.float32),
    compiler_params=_sc_params,
    scratch_types=[
        pltpu.VMEM((NCH, CH), jnp.int32),
        pltpu.VMEM((CH,), jnp.float32),
        pltpu.VMEM((NSL,), jnp.float32),
        pltpu.VMEM_SHARED((NPAD,), jnp.float32),
        pltpu.SemaphoreType.DMA,
    ],
)
def _deg_kernel(edge_hbm, out_hbm, dst_v, ones_v, z_v, deg_sh, dsem):
    c = lax.axis_index("c")
    s = lax.axis_index("s")
    w = c * NTILE + s
    pltpu.sync_copy(edge_hbm.at[1, w], dst_v)

    def permute(i, _):
        j = i >> 3
        kk = i & 7
        v = dst_v[j, pl.ds(kk * 16, 16)]
        dst_v[j, pl.ds(kk * 16, 16)] = (v & 3) * KP + (v >> 2)
        return 0

    lax.fori_loop(0, NCH * CH // 16, permute, 0)

    def fill_ones(i, _):
        ones_v[pl.ds(i * 16, 16)] = jnp.ones((16,), jnp.float32)
        return 0

    lax.fori_loop(0, CH // 16, fill_ones, 0)

    def fill_zero(i, _):
        z_v[pl.ds(i * 16, 16)] = _Z16()
        return 0

    lax.fori_loop(0, NSL // 16, fill_zero, 0)
    pltpu.sync_copy(z_v, deg_sh.at[pl.ds(s * NSL, NSL)])
    plsc.subcore_barrier()

    def group(g, _):
        j0 = g * 8
        for q in range(8):
            pltpu.async_copy(ones_v, deg_sh.at[dst_v.at[j0 + q]], dsem,
                             add=True)
        for q in range(8):
            pltpu.make_async_copy(ones_v, deg_sh.at[dst_v.at[j0 + q]],
                                  dsem).wait()
        return 0

    lax.fori_loop(0, NCH // 8, group, 0)
    plsc.subcore_barrier()
    pltpu.sync_copy(deg_sh.at[pl.ds(s * NSL, NSL)],
                    out_hbm.at[c, pl.ds(s * NSL, NSL)])


@functools.partial(
    pl.kernel,
    mesh=_mesh,
    out_type=jax.ShapeDtypeStruct((NSC, NPAD, H), jnp.float32),
    compiler_params=_sc_params,
    scratch_types=[
        pltpu.VMEM((NCH, CH), jnp.int32),
        pltpu.VMEM((NCH, CH), jnp.int32),
        *([pltpu.VMEM((CH, H), jnp.float32)] * 8),
        pltpu.VMEM((NSL, H), jnp.float32),
        pltpu.VMEM_SHARED((NPAD, H), jnp.float32),
        *([pltpu.SemaphoreType.DMA] * 16),
    ],
)
def _agg_kernel(edge_hbm, hp_hbm, out_hbm, src_v, dst_v, *bufs):
    rows = bufs[0:8]
    zb_v = bufs[8]
    agg_sh = bufs[9]
    gsem = bufs[10:18]
    ssem = bufs[18:26]
    c = lax.axis_index("c")
    s = lax.axis_index("s")
    w = c * NTILE + s
    pltpu.sync_copy(edge_hbm.at[0, w], src_v)
    pltpu.sync_copy(edge_hbm.at[1, w], dst_v)

    def fill_zero(i, _):
        zb_v[i, pl.ds(0, 16)] = _Z16()
        zb_v[i, pl.ds(16, 16)] = _Z16()
        return 0

    lax.fori_loop(0, NSL, fill_zero, 0)
    pltpu.sync_copy(zb_v, agg_sh.at[pl.ds(s * NSL, NSL)])
    plsc.subcore_barrier()

    NB = 8
    for p in range(NB - 1):
        pltpu.async_copy(hp_hbm.at[src_v.at[p]], rows[p], gsem[p])

    def step(i, _):
        j0 = i * NB
        for b in range(NB):
            j = j0 + b
            nxt = j + NB - 1
            nb = (b + NB - 1) % NB

            @pl.when(nxt < NCH)
            def _():
                @pl.when(j >= 1)
                def _():
                    pltpu.make_async_copy(rows[nb], agg_sh.at[dst_v.at[j - 1]],
                                          ssem[nb]).wait()
                pltpu.async_copy(hp_hbm.at[src_v.at[nxt]], rows[nb], gsem[nb])

            pltpu.make_async_copy(hp_hbm.at[src_v.at[j]],
                                  rows[b], gsem[b]).wait()
            pltpu.async_copy(rows[b], agg_sh.at[dst_v.at[j]], ssem[b],
                             add=True)
        return 0

    lax.fori_loop(0, NCH // NB, step, 0)
    for b in range(NB):
        j = NCH - NB + b
        pltpu.make_async_copy(rows[b], agg_sh.at[dst_v.at[j]],
                              ssem[b]).wait()
    plsc.subcore_barrier()
    pltpu.sync_copy(agg_sh.at[pl.ds(s * NSL, NSL)],
                    out_hbm.at[c, pl.ds(s * NSL, NSL)])


KP = NPAD // 4


def _scale_cols(degp4):
    deg4 = degp4[0] + degp4[1]
    kpb = deg4.shape[1]
    r = lax.broadcasted_iota(jnp.int32, (4, 4), 0)
    c = lax.broadcasted_iota(jnp.int32, (4, 4), 1)
    eye4 = (r == c).astype(jnp.float32)
    deg_n = lax.dot_general(deg4, eye4, (((0,), (0,)), ((), ())),
                            preferred_element_type=jnp.float32)
    return lax.rsqrt(jnp.maximum(deg_n, 1.0))


KPB = KP // 4


def _hmat_body(x_ref, w1_ref, out_ref):
    for o in range(4):
        x_o = x_ref[:, o * DF:(o + 1) * DF]
        out_ref[:, o * H:(o + 1) * H] = jnp.dot(
            x_o, w1_ref[...], preferred_element_type=jnp.float32)


def _hmat_call(x_r, w1):
    return pl.pallas_call(
        _hmat_body,
        grid=(4,),
        in_specs=[
            pl.BlockSpec((KPB, 4 * DF), lambda i: (i, 0)),
            pl.BlockSpec((DF, H), lambda i: (0, 0)),
        ],
        out_specs=pl.BlockSpec((KPB, 4 * H), lambda i: (i, 0)),
        out_shape=jax.ShapeDtypeStruct((KP, 4 * H), jnp.float32),
    )(x_r, w1)


def _hscale_body(h_ref, degp_ref, out_ref):
    scale = _scale_cols(degp_ref[...])
    for o in range(4):
        out_ref[:, o * H:(o + 1) * H] = (
            h_ref[:, o * H:(o + 1) * H] * scale[:, o:o + 1])


def _hscale_call(h_p, degp4):
    return pl.pallas_call(
        _hscale_body,
        out_shape=jax.ShapeDtypeStruct((KP, 4 * H), jnp.float32),
    )(h_p, degp4)


def _tail_body(aggp_ref, degp_ref, b1_ref, ind_ref, w2_ref, b2_ref,
               w3_ref, b3_ref, out_ref):
    scale = _scale_cols(degp_ref[...])
    agg = aggp_ref[0] + aggp_ref[1]
    gids = lax.broadcasted_iota(jnp.int32, (NG, KP), 0)
    sums = jnp.zeros((NG, H), jnp.float32)
    counts = jnp.zeros((NG, 1), jnp.float32)
    for o in range(4):
        h2_o = jnp.maximum(
            agg[:, o * H:(o + 1) * H] * scale[:, o:o + 1] + b1_ref[...], 0.0)
        onehot_o = (ind_ref[o:o + 1, :] == gids).astype(jnp.float32)
        sums = sums + jnp.dot(onehot_o, h2_o,
                              preferred_element_type=jnp.float32)
        counts = counts + jnp.sum(onehot_o, axis=1, keepdims=True)
    pooled = sums / jnp.maximum(counts, 1.0)
    z = jnp.maximum(
        jnp.dot(pooled, w2_ref[...], preferred_element_type=jnp.float32)
        + b2_ref[...], 0.0)
    logits = (jnp.dot(z, w3_ref[...], preferred_element_type=jnp.float32)
              + b3_ref[...])
    m = jnp.max(logits, axis=-1, keepdims=True)
    ex = jnp.exp(logits - m)
    out_ref[...] = ex / jnp.sum(ex, axis=-1, keepdims=True)


def _tail_call(aggp, degp_r, b1, ind, w2, b2, w3, b3):
    return pl.pallas_call(
        _tail_body,
        out_shape=jax.ShapeDtypeStruct((NG, NCLS), jnp.float32),
    )(aggp, degp_r, b1, ind, w2, b2, w3, b3)


def kernel(x, edge_index, node_indicator, W1, b1, W2, b2, W3, b3):
    ei = edge_index.astype(jnp.int32)
    npe = EPAD - E
    pad_src = (jnp.arange(npe, dtype=jnp.int32) * 97) % N
    pad_dst = N + (jnp.arange(npe, dtype=jnp.int32) % (NPAD - N))
    edge4 = jnp.concatenate(
        [ei, jnp.stack([pad_src, pad_dst])], axis=1).reshape(2, NW, NCH, CH)

    degp = _deg_kernel(edge4)
    degp4 = degp.reshape(NSC, 4, KP)
    x_r = jnp.pad(x, ((0, NPAD - N), (0, 0))).reshape(KP, 4 * DF)
    h_p = _hmat_call(x_r, W1)
    hp_p = _hscale_call(h_p, degp4)
    hp = hp_p.reshape(NPAD, H)
    aggp = _agg_kernel(edge4, hp)
    aggp_r = aggp.reshape(NSC, KP, 4 * H)

    ind = jnp.concatenate(
        [node_indicator.astype(jnp.int32),
         jnp.full((NPAD - N,), NG, jnp.int32)])
    ind4 = ind.reshape(KP, 4).T
    return _tail_call(aggp_r, degp4, b1.reshape(1, H), ind4,
                      W2, b2.reshape(1, DENSE), W3, b3.reshape(1, NCLS))
